# baseline (speedup 1.0000x reference)
import functools

import jax
import jax.numpy as jnp
from jax import lax
from jax.experimental import pallas as pl
from jax.experimental.pallas import tpu as pltpu
from jax.experimental.pallas import tpu_sc as plsc

NC = 2
NS = 16
NW = NC * NS
_RSQRT_MAGIC = 0x5F3759DF


def _pad_up(v, m):
    return (v + m - 1) // m * m


def _vfill(ref, n, val):
    v = jnp.full((16,), val, jnp.float32)

    def body(j, carry):
        ref[pl.ds(j * 16, 16)] = v
        return carry

    lax.fori_loop(0, n // 16, body, 0)


def _edge_pass(src_hbm, dst_hbm, tabs, accs, msgs, isrc, idst,
               sem_i, sem_g, sem_s, base, nch, CH, B):
    def load_idx(k, slot):
        off = pl.multiple_of(base + k * CH, 8)
        pltpu.async_copy(src_hbm.at[pl.ds(off, CH)], isrc[slot], sem_i)
        pltpu.async_copy(dst_hbm.at[pl.ds(off, CH)], idst[slot], sem_i)

    def wait_idx(k, slot):
        off = pl.multiple_of(base + k * CH, 8)
        pltpu.make_async_copy(
            src_hbm.at[pl.ds(off, CH)], isrc[slot], sem_i).wait()
        pltpu.make_async_copy(
            dst_hbm.at[pl.ds(off, CH)], idst[slot], sem_i).wait()

    def drain_scatters(slot):
        for b in range(B):
            pltpu.make_async_copy(msgs[b][slot],
                                  accs[b].at[idst[slot]], sem_s).wait()

    load_idx(0, 0)

    def pair(k2, carry):
        for p in (0, 1):
            q = 1 - p
            k = k2 * 2 + p
            wait_idx(k, p)
            gs = [pltpu.async_copy(tabs[b].at[isrc[p]], msgs[b][p], sem_g)
                  for b in range(B)]
            if p == 0:
                @pl.when(k2 > 0)
                def _():
                    drain_scatters(q)

                load_idx(k + 1, q)
            else:
                drain_scatters(q)

                @pl.when(k2 + 1 < nch // 2)
                def _():
                    load_idx(k + 1, q)

            for g in gs:
                g.wait()
            for b in range(B):
                pltpu.async_copy(msgs[b][p], accs[b].at[idst[p]], sem_s,
                                 add=True)
        return carry

    lax.fori_loop(0, nch // 2, pair, 0)
    drain_scatters(1)


@functools.lru_cache(maxsize=None)
def _edge1_kernel(B, Np, Ep, CH):
    SLN = Np // NS
    nch = Ep // (NW * CH)
    nch_h = Ep // (NS * CH)
    ew = nch * CH
    mesh = plsc.VectorSubcoreMesh(
        core_axis_name="c", subcore_axis_name="s", num_cores=NC, num_subcores=NS)

    scratch = ([pltpu.VMEM_SHARED((Np,), jnp.float32)]
               + [pltpu.VMEM_SHARED((Np,), jnp.float32)] * B
               + [pltpu.VMEM_SHARED((Np,), jnp.float32)] * B
               + [pltpu.VMEM((CH,), jnp.float32)] * (2 * B)
               + [pltpu.VMEM((SLN,), jnp.float32)] * B
               + [pltpu.VMEM((CH,), jnp.int32)] * 2
               + [pltpu.VMEM((CH,), jnp.int32)] * 2
               + [pltpu.VMEM((CH,), jnp.float32),
                  pltpu.VMEM((SLN,), jnp.float32),
                  pltpu.VMEM((SLN,), jnp.float32),
                  pltpu.SemaphoreType.DMA,
                  pltpu.SemaphoreType.DMA,
                  pltpu.SemaphoreType.DMA,
                  pltpu.SemaphoreType.DMA])

    @functools.partial(
        pl.kernel,
        out_type=(jax.ShapeDtypeStruct((NC * B * Np,), jnp.float32),
                  jax.ShapeDtypeStruct((Np,), jnp.float32)),
        mesh=mesh,
        scratch_types=scratch,
    )
    def edge1(x_hbm, src_hbm, dst_hbm, ones_hbm, out_hbm, dinv_hbm, *refs):
        dega = refs[0]
        tabs = refs[1:1 + B]
        accs = refs[1 + B:1 + 2 * B]
        mflat = refs[1 + 2 * B:1 + 4 * B]
        msgs = [(mflat[2 * b], mflat[2 * b + 1]) for b in range(B)]
        xbufs = refs[1 + 4 * B:1 + 5 * B]
        (isrc0, isrc1, idst0, idst1, ones_v, degv, bounce,
         sem_i, sem_x, sem_g, sem_s) = refs[1 + 5 * B:]
        isrc = (isrc0, isrc1)
        idst = (idst0, idst1)
        c = lax.axis_index("c")
        s = lax.axis_index("s")
        wid = c * NS + s
        row = pl.multiple_of(s * SLN, 8)
        rsl = pl.ds(row, SLN)

        _vfill(bounce, SLN, 0.0)
        pltpu.sync_copy(ones_hbm, ones_v)
        pltpu.sync_copy(bounce, dega.at[rsl])
        for b in range(B):
            pltpu.sync_copy(bounce, accs[b].at[rsl])
        xds = [pltpu.async_copy(
            x_hbm.at[pl.ds(pl.multiple_of(b * Np + row, 8), SLN)],
            xbufs[b], sem_x) for b in range(B)]
        plsc.subcore_barrier()

        hbase = s * (nch_h * CH)

        def hload(k, slot):
            off = pl.multiple_of(hbase + k * CH, 8)
            pltpu.async_copy(dst_hbm.at[pl.ds(off, CH)], idst[slot], sem_i)

        def hwait(k, slot):
            off = pl.multiple_of(hbase + k * CH, 8)
            pltpu.make_async_copy(
                dst_hbm.at[pl.ds(off, CH)], idst[slot], sem_i).wait()

        def hdrain(slot):
            pltpu.make_async_copy(ones_v, dega.at[idst[slot]], sem_s).wait()

        hload(0, 0)

        def hpair(k2, carry):
            for p in (0, 1):
                q = 1 - p
                k = k2 * 2 + p
                hwait(k, p)
                if p == 0:
                    @pl.when(k2 > 0)
                    def _():
                        hdrain(q)

                    hload(k + 1, q)
                else:
                    hdrain(q)

                    @pl.when(k2 + 1 < nch_h // 2)
                    def _():
                        hload(k + 1, q)

                pltpu.async_copy(ones_v, dega.at[idst[p]], sem_s, add=True)
            return carry

        lax.fori_loop(0, nch_h // 2, hpair, 0)
        hdrain(1)
        plsc.subcore_barrier()

        pltpu.sync_copy(dega.at[rsl], degv)
        for d in xds:
            d.wait()
        magic = jnp.full((16,), _RSQRT_MAGIC, jnp.int32)

        def dense(j, carry):
            sl = pl.ds(j * 16, 16)
            d = degv[sl] + 1.0
            yi = magic - lax.shift_right_logical(
                lax.bitcast_convert_type(d, jnp.int32), 1)
            yv = lax.bitcast_convert_type(yi, jnp.float32)
            for _ in range(3):
                yv = yv * (1.5 - 0.5 * d * yv * yv)
            degv[sl] = yv
            for b in range(B):
                xbufs[b][sl] = xbufs[b][sl] * yv
            return carry

        lax.fori_loop(0, SLN // 16, dense, 0)
        for b in range(B):
            pltpu.sync_copy(xbufs[b], tabs[b].at[rsl])

        @pl.when(c == 0)
        def _():
            pltpu.sync_copy(degv, dinv_hbm.at[rsl])

        plsc.subcore_barrier()

        _edge_pass(src_hbm, dst_hbm, tabs, accs, msgs, isrc, idst,
                   sem_i, sem_g, sem_s, wid * ew, nch, CH, B)
        plsc.subcore_barrier()
        for b in range(B):
            pltpu.sync_copy(accs[b].at[rsl], bounce)
            orow = pl.multiple_of((c * B + b) * Np + row, 8)
            pltpu.sync_copy(bounce, out_hbm.at[pl.ds(orow, SLN)])

    return edge1


@functools.lru_cache(maxsize=None)
def _edge2_kernel(B, Np, Ep, CH):
    SLN = Np // NS
    nch = Ep // (NW * CH)
    ew = nch * CH
    mesh = plsc.VectorSubcoreMesh(
        core_axis_name="c", subcore_axis_name="s", num_cores=NC, num_subcores=NS)

    scratch = ([pltpu.VMEM_SHARED((Np,), jnp.float32)] * B
               + [pltpu.VMEM_SHARED((Np,), jnp.float32)] * B
               + [pltpu.VMEM((CH,), jnp.float32)] * (2 * B)
               + [pltpu.VMEM((CH,), jnp.int32)] * 2
               + [pltpu.VMEM((CH,), jnp.int32)] * 2
               + [pltpu.VMEM((SLN,), jnp.float32),
                  pltpu.SemaphoreType.DMA,
                  pltpu.SemaphoreType.DMA,
                  pltpu.SemaphoreType.DMA])

    @functools.partial(
        pl.kernel,
        out_type=jax.ShapeDtypeStruct((NC * B * Np,), jnp.float32),
        mesh=mesh,
        scratch_types=scratch,
    )
    def edge2(y_hbm, src_hbm, dst_hbm, out_hbm, *refs):
        tabs = refs[:B]
        accs = refs[B:2 * B]
        mflat = refs[2 * B:4 * B]
        msgs = [(mflat[2 * b], mflat[2 * b + 1]) for b in range(B)]
        (isrc0, isrc1, idst0, idst1, bounce,
         sem_i, sem_g, sem_s) = refs[4 * B:]
        isrc = (isrc0, isrc1)
        idst = (idst0, idst1)
        c = lax.axis_index("c")
        s = lax.axis_index("s")
        wid = c * NS + s
        row = pl.multiple_of(s * SLN, 8)
        rsl = pl.ds(row, SLN)

        _vfill(bounce, SLN, 0.0)
        for b in range(B):
            pltpu.sync_copy(bounce, accs[b].at[rsl])
        for b in range(B):
            yrow = pl.multiple_of(b * Np + row, 8)
            pltpu.sync_copy(y_hbm.at[pl.ds(yrow, SLN)], bounce)
            pltpu.sync_copy(bounce, tabs[b].at[rsl])
        plsc.subcore_barrier()

        _edge_pass(src_hbm, dst_hbm, tabs, accs, msgs, isrc, idst,
                   sem_i, sem_g, sem_s, wid * ew, nch, CH, B)
        plsc.subcore_barrier()
        for b in range(B):
            pltpu.sync_copy(accs[b].at[rsl], bounce)
            orow = pl.multiple_of((c * B + b) * Np + row, 8)
            pltpu.sync_copy(bounce, out_hbm.at[pl.ds(orow, SLN)])

    return edge2


def _dense_mid(aggp, dinv, xp, w1, b1, w2):
    H = w1.shape[0]
    B, Np = xp.shape
    CB = 3584
    assert Np % CB == 0

    def body(aggp_ref, dinv_ref, xp_ref, w1_ref, b1_ref, w2_ref, t_ref):
        dv = dinv_ref[...]
        a = (aggp_ref[0] + aggp_ref[1] + xp_ref[...] * dv) * dv

        def jb(j, acc):
            return acc + jnp.maximum(a * w1_ref[j] + b1_ref[j], 0.0) * w2_ref[j]

        sv = lax.fori_loop(0, H, jb, jnp.zeros_like(a))
        t_ref[...] = sv * dv

    smem = pl.BlockSpec(memory_space=pltpu.SMEM)
    return pl.pallas_call(
        body,
        grid=(Np // CB,),
        in_specs=[pl.BlockSpec((NC, B, CB), lambda i: (0, 0, i)),
                  pl.BlockSpec((1, CB), lambda i: (0, i)),
                  pl.BlockSpec((B, CB), lambda i: (0, i)),
                  smem, smem, smem],
        out_specs=pl.BlockSpec((B, CB), lambda i: (0, i)),
        out_shape=jax.ShapeDtypeStruct(xp.shape, jnp.float32),
    )(aggp, dinv, xp, w1, b1, w2)


def _dense_fini(outp, dinv, t, b2):
    def body(outp_ref, dinv_ref, t_ref, b2_ref, o_ref):
        dv = dinv_ref[...]
        o_ref[...] = (outp_ref[0] + outp_ref[1] + t_ref[...]) * dv + b2_ref[0]

    smem = pl.BlockSpec(memory_space=pltpu.SMEM)
    vmem = pl.BlockSpec(memory_space=pltpu.VMEM)
    return pl.pallas_call(
        body,
        in_specs=[vmem, vmem, vmem, smem],
        out_shape=jax.ShapeDtypeStruct(t.shape, jnp.float32),
    )(outp, dinv, t, b2)


def kernel(x, edge_index, W1, b1, W2, b2):
    B, N = x.shape
    E = edge_index.shape[1]
    Np = (N // 256 + 1) * 256
    CH = 2504

    Ep = _pad_up(E, NW * CH * 2)
    if Ep != E:
        pad = jnp.full((2, Ep - E), N, dtype=edge_index.dtype)
        e = jnp.concatenate([edge_index, pad], axis=1)
    else:
        e = edge_index
    src = e[0].reshape(-1)
    dst = e[1].reshape(-1)

    xp = jnp.pad(x.astype(jnp.float32), ((0, 0), (0, Np - N)))

    ones = jnp.ones((CH,), jnp.float32)
    aggp, dinv = _edge1_kernel(B, Np, Ep, CH)(xp.reshape(-1), src, dst, ones)
    aggp = aggp.reshape(NC, B, Np)
    t = _dense_mid(aggp, dinv.reshape(1, Np), xp, W1.reshape(-1),
                   b1.reshape(-1), W2.reshape(-1))
    outp = _edge2_kernel(B, Np, Ep, CH)(
        t.reshape(-1), src, dst).reshape(NC, B, Np)
    o = _dense_fini(outp, dinv.reshape(1, Np), t, b2.reshape(-1))
    return o[:, :N]


# speedup vs baseline: 1.1167x; 1.1167x over previous
import functools

import jax
import jax.numpy as jnp
from jax import lax
from jax.experimental import pallas as pl
from jax.experimental.pallas import tpu as pltpu
from jax.experimental.pallas import tpu_sc as plsc

NC = 2
NS = 16
NW = NC * NS
_RSQRT_MAGIC = 0x5F3759DF


def _pad_up(v, m):
    return (v + m - 1) // m * m


def _vfill(ref, n, val):
    v = jnp.full((16,), val, jnp.float32)

    def body(j, carry):
        ref[pl.ds(j * 16, 16)] = v
        return carry

    lax.fori_loop(0, n // 16, body, 0)


def _edge_pass(src_hbm, dst_hbm, tabs, accs, msgs, isrc, idst,
               sem_i, sem_g, sem_s, base, nch, CH, B):
    def load_idx(k, slot):
        off = pl.multiple_of(base + k * CH, 8)
        pltpu.async_copy(src_hbm.at[pl.ds(off, CH)], isrc[slot], sem_i)
        pltpu.async_copy(dst_hbm.at[pl.ds(off, CH)], idst[slot], sem_i)

    def wait_idx(k, slot):
        off = pl.multiple_of(base + k * CH, 8)
        pltpu.make_async_copy(
            src_hbm.at[pl.ds(off, CH)], isrc[slot], sem_i).wait()
        pltpu.make_async_copy(
            dst_hbm.at[pl.ds(off, CH)], idst[slot], sem_i).wait()

    def drain_scatters(slot):
        for b in range(B):
            pltpu.make_async_copy(msgs[b][slot],
                                  accs[b].at[idst[slot]], sem_s).wait()

    load_idx(0, 0)

    def pair(k2, carry):
        for p in (0, 1):
            q = 1 - p
            k = k2 * 2 + p
            wait_idx(k, p)
            gs = [pltpu.async_copy(tabs[b].at[isrc[p]], msgs[b][p], sem_g)
                  for b in range(B)]
            if p == 0:
                @pl.when(k2 > 0)
                def _():
                    drain_scatters(q)

                load_idx(k + 1, q)
            else:
                drain_scatters(q)

                @pl.when(k2 + 1 < nch // 2)
                def _():
                    load_idx(k + 1, q)

            for g in gs:
                g.wait()
            for b in range(B):
                pltpu.async_copy(msgs[b][p], accs[b].at[idst[p]], sem_s,
                                 add=True)
        return carry

    lax.fori_loop(0, nch // 2, pair, 0)
    drain_scatters(1)


@functools.lru_cache(maxsize=None)
def _edge1_kernel(B, Np, Ep, CH):
    SLN = Np // NS
    nch = Ep // (NW * CH)
    nch_h = Ep // (NS * CH)
    ew = nch * CH
    mesh = plsc.VectorSubcoreMesh(
        core_axis_name="c", subcore_axis_name="s", num_cores=NC, num_subcores=NS)

    scratch = ([pltpu.VMEM_SHARED((Np,), jnp.float32)]
               + [pltpu.VMEM_SHARED((Np,), jnp.float32)] * B
               + [pltpu.VMEM_SHARED((Np,), jnp.float32)] * B
               + [pltpu.VMEM((CH,), jnp.float32)] * (2 * B)
               + [pltpu.VMEM((SLN,), jnp.float32)] * B
               + [pltpu.VMEM((CH,), jnp.int32)] * 2
               + [pltpu.VMEM((CH,), jnp.int32)] * 2
               + [pltpu.VMEM((CH,), jnp.float32),
                  pltpu.VMEM((SLN,), jnp.float32),
                  pltpu.VMEM((SLN,), jnp.float32),
                  pltpu.SemaphoreType.DMA,
                  pltpu.SemaphoreType.DMA,
                  pltpu.SemaphoreType.DMA,
                  pltpu.SemaphoreType.DMA])

    @functools.partial(
        pl.kernel,
        out_type=(jax.ShapeDtypeStruct((NC * B * Np,), jnp.float32),
                  jax.ShapeDtypeStruct((Np,), jnp.float32)),
        mesh=mesh,
        scratch_types=scratch,
    )
    def edge1(x_hbm, src_hbm, dst_hbm, ones_hbm, out_hbm, dinv_hbm, *refs):
        dega = refs[0]
        tabs = refs[1:1 + B]
        accs = refs[1 + B:1 + 2 * B]
        mflat = refs[1 + 2 * B:1 + 4 * B]
        msgs = [(mflat[2 * b], mflat[2 * b + 1]) for b in range(B)]
        xbufs = refs[1 + 4 * B:1 + 5 * B]
        (isrc0, isrc1, idst0, idst1, ones_v, degv, bounce,
         sem_i, sem_x, sem_g, sem_s) = refs[1 + 5 * B:]
        isrc = (isrc0, isrc1)
        idst = (idst0, idst1)
        c = lax.axis_index("c")
        s = lax.axis_index("s")
        wid = c * NS + s
        row = pl.multiple_of(s * SLN, 8)
        rsl = pl.ds(row, SLN)

        _vfill(bounce, SLN, 0.0)
        pltpu.sync_copy(ones_hbm, ones_v)
        pltpu.sync_copy(bounce, dega.at[rsl])
        for b in range(B):
            pltpu.sync_copy(bounce, accs[b].at[rsl])
        xds = [pltpu.async_copy(
            x_hbm.at[pl.ds(pl.multiple_of(b * Np + row, 8), SLN)],
            xbufs[b], sem_x) for b in range(B)]
        plsc.subcore_barrier()

        hbase = s * (nch_h * CH)

        def hload(k, slot):
            off = pl.multiple_of(hbase + k * CH, 8)
            pltpu.async_copy(dst_hbm.at[pl.ds(off, CH)], idst[slot], sem_i)

        def hwait(k, slot):
            off = pl.multiple_of(hbase + k * CH, 8)
            pltpu.make_async_copy(
                dst_hbm.at[pl.ds(off, CH)], idst[slot], sem_i).wait()

        def hdrain(slot):
            pltpu.make_async_copy(ones_v, dega.at[idst[slot]], sem_s).wait()

        hload(0, 0)

        def hpair(k2, carry):
            for p in (0, 1):
                q = 1 - p
                k = k2 * 2 + p
                hwait(k, p)
                if p == 0:
                    @pl.when(k2 > 0)
                    def _():
                        hdrain(q)

                    hload(k + 1, q)
                else:
                    hdrain(q)

                    @pl.when(k2 + 1 < nch_h // 2)
                    def _():
                        hload(k + 1, q)

                pltpu.async_copy(ones_v, dega.at[idst[p]], sem_s, add=True)
            return carry

        lax.fori_loop(0, nch_h // 2, hpair, 0)
        hdrain(1)
        plsc.subcore_barrier()

        pltpu.sync_copy(dega.at[rsl], degv)
        for d in xds:
            d.wait()
        magic = jnp.full((16,), _RSQRT_MAGIC, jnp.int32)

        def dense(j, carry):
            sl = pl.ds(j * 16, 16)
            d = degv[sl] + 1.0
            yi = magic - lax.shift_right_logical(
                lax.bitcast_convert_type(d, jnp.int32), 1)
            yv = lax.bitcast_convert_type(yi, jnp.float32)
            for _ in range(3):
                yv = yv * (1.5 - 0.5 * d * yv * yv)
            degv[sl] = yv
            for b in range(B):
                xbufs[b][sl] = xbufs[b][sl] * yv
            return carry

        lax.fori_loop(0, SLN // 16, dense, 0)
        for b in range(B):
            pltpu.sync_copy(xbufs[b], tabs[b].at[rsl])

        @pl.when(c == 0)
        def _():
            pltpu.sync_copy(degv, dinv_hbm.at[rsl])

        plsc.subcore_barrier()

        _edge_pass(src_hbm, dst_hbm, tabs, accs, msgs, isrc, idst,
                   sem_i, sem_g, sem_s, wid * ew, nch, CH, B)
        plsc.subcore_barrier()
        for b in range(B):
            pltpu.sync_copy(accs[b].at[rsl], bounce)
            orow = pl.multiple_of((c * B + b) * Np + row, 8)
            pltpu.sync_copy(bounce, out_hbm.at[pl.ds(orow, SLN)])

    return edge1


@functools.lru_cache(maxsize=None)
def _edge2_kernel(B, Np, Ep, CH):
    SLN = Np // NS
    nch = Ep // (NW * CH)
    ew = nch * CH
    mesh = plsc.VectorSubcoreMesh(
        core_axis_name="c", subcore_axis_name="s", num_cores=NC, num_subcores=NS)

    scratch = ([pltpu.VMEM_SHARED((Np,), jnp.float32)] * B
               + [pltpu.VMEM_SHARED((Np,), jnp.float32)] * B
               + [pltpu.VMEM((CH,), jnp.float32)] * (2 * B)
               + [pltpu.VMEM((CH,), jnp.int32)] * 2
               + [pltpu.VMEM((CH,), jnp.int32)] * 2
               + [pltpu.VMEM((SLN,), jnp.float32),
                  pltpu.SemaphoreType.DMA,
                  pltpu.SemaphoreType.DMA,
                  pltpu.SemaphoreType.DMA])

    @functools.partial(
        pl.kernel,
        out_type=jax.ShapeDtypeStruct((NC * B * Np,), jnp.float32),
        mesh=mesh,
        scratch_types=scratch,
    )
    def edge2(y_hbm, src_hbm, dst_hbm, out_hbm, *refs):
        tabs = refs[:B]
        accs = refs[B:2 * B]
        mflat = refs[2 * B:4 * B]
        msgs = [(mflat[2 * b], mflat[2 * b + 1]) for b in range(B)]
        (isrc0, isrc1, idst0, idst1, bounce,
         sem_i, sem_g, sem_s) = refs[4 * B:]
        isrc = (isrc0, isrc1)
        idst = (idst0, idst1)
        c = lax.axis_index("c")
        s = lax.axis_index("s")
        wid = c * NS + s
        row = pl.multiple_of(s * SLN, 8)
        rsl = pl.ds(row, SLN)

        _vfill(bounce, SLN, 0.0)
        for b in range(B):
            pltpu.sync_copy(bounce, accs[b].at[rsl])
        for b in range(B):
            yrow = pl.multiple_of(b * Np + row, 8)
            pltpu.sync_copy(y_hbm.at[pl.ds(yrow, SLN)], bounce)
            pltpu.sync_copy(bounce, tabs[b].at[rsl])
        plsc.subcore_barrier()

        _edge_pass(src_hbm, dst_hbm, tabs, accs, msgs, isrc, idst,
                   sem_i, sem_g, sem_s, wid * ew, nch, CH, B)
        plsc.subcore_barrier()
        for b in range(B):
            pltpu.sync_copy(accs[b].at[rsl], bounce)
            orow = pl.multiple_of((c * B + b) * Np + row, 8)
            pltpu.sync_copy(bounce, out_hbm.at[pl.ds(orow, SLN)])

    return edge2


def _dense_mid(aggp, dinv, xp, w1, b1, w2):
    H = w1.shape[0]
    B, Np = xp.shape
    CB = 3584
    assert Np % CB == 0

    def body(aggp_ref, dinv_ref, xp_ref, w1_ref, b1_ref, w2_ref, t_ref):
        dv = dinv_ref[...]
        a = (aggp_ref[0] + aggp_ref[1] + xp_ref[...] * dv) * dv

        def jb(j, acc):
            return acc + jnp.maximum(a * w1_ref[j] + b1_ref[j], 0.0) * w2_ref[j]

        sv = lax.fori_loop(0, H, jb, jnp.zeros_like(a))
        t_ref[...] = sv * dv

    smem = pl.BlockSpec(memory_space=pltpu.SMEM)
    return pl.pallas_call(
        body,
        grid=(Np // CB,),
        in_specs=[pl.BlockSpec((NC, B, CB), lambda i: (0, 0, i)),
                  pl.BlockSpec((1, CB), lambda i: (0, i)),
                  pl.BlockSpec((B, CB), lambda i: (0, i)),
                  smem, smem, smem],
        out_specs=pl.BlockSpec((B, CB), lambda i: (0, i)),
        out_shape=jax.ShapeDtypeStruct(xp.shape, jnp.float32),
    )(aggp, dinv, xp, w1, b1, w2)


def _dense_fini(outp, dinv, t, b2):
    def body(outp_ref, dinv_ref, t_ref, b2_ref, o_ref):
        dv = dinv_ref[...]
        o_ref[...] = (outp_ref[0] + outp_ref[1] + t_ref[...]) * dv + b2_ref[0]

    smem = pl.BlockSpec(memory_space=pltpu.SMEM)
    vmem = pl.BlockSpec(memory_space=pltpu.VMEM)
    return pl.pallas_call(
        body,
        in_specs=[vmem, vmem, vmem, smem],
        out_shape=jax.ShapeDtypeStruct(t.shape, jnp.float32),
    )(outp, dinv, t, b2)


def kernel(x, edge_index, W1, b1, W2, b2):
    B, N = x.shape
    E = edge_index.shape[1]
    Np = (N // 256 + 1) * 256
    CH = 6256

    Ep = _pad_up(E, NW * CH * 2)
    if Ep != E:
        pad = jnp.full((2, Ep - E), N, dtype=edge_index.dtype)
        e = jnp.concatenate([edge_index, pad], axis=1)
    else:
        e = edge_index
    src = e[0].reshape(-1)
    dst = e[1].reshape(-1)

    xp = jnp.pad(x.astype(jnp.float32), ((0, 0), (0, Np - N)))

    ones = jnp.ones((CH,), jnp.float32)
    aggp, dinv = _edge1_kernel(B, Np, Ep, CH)(xp.reshape(-1), src, dst, ones)
    aggp = aggp.reshape(NC, B, Np)
    t = _dense_mid(aggp, dinv.reshape(1, Np), xp, W1.reshape(-1),
                   b1.reshape(-1), W2.reshape(-1))
    outp = _edge2_kernel(B, Np, Ep, CH)(
        t.reshape(-1), src, dst).reshape(NC, B, Np)
    o = _dense_fini(outp, dinv.reshape(1, Np), t, b2.reshape(-1))
    return o[:, :N]
